# baseline (speedup 1.0000x reference)
import functools

import jax
import jax.numpy as jnp
from jax.experimental import pallas as pl
from jax.experimental.pallas import tpu as pltpu

_N = 512
_H = 64
_EA = 4
_REG = 0.01
_EPS = 1e-8
_BI = 16


def _silu(v):
    return v * jax.nn.sigmoid(v)


def _egcl_kernel(h_i_ref, h_all_ref, x_i_ref, x_all_ref, ea_ref,
                 w1r_ref, w1c_ref, w1d_ref, w1e_ref, be1_ref,
                 we2_ref, be2_ref, wc1_ref, bc1_ref, wc2_ref, bc2_ref,
                 wn1h_ref, wn1m_ref, bn1_ref, wn2_ref, bn2_ref,
                 h_out_ref, x_out_ref):
    f32 = jnp.float32
    h_i = h_i_ref[0]
    h_all = h_all_ref[0]
    x_i = x_i_ref[0]
    x_all = x_all_ref[0]

    hA = jnp.dot(h_i, w1r_ref[...], preferred_element_type=f32)
    hB = jnp.dot(h_all, w1c_ref[...], preferred_element_type=f32)

    d = [x_i[:, k:k + 1][:, :, None] - x_all[:, k:k + 1][None, :, :]
         for k in range(3)]
    rad = (d[0] * d[0] + d[1] * d[1]) + d[2] * d[2]

    ea = ea_ref[0]
    z1 = (hA[:, None, :] + hB[None, :, :]
          + rad * w1d_ref[...][None, :, :]
          + be1_ref[...][None, :, :])
    for k in range(_EA):
        z1 = z1 + ea[:, :, k:k + 1] * w1e_ref[k:k + 1, :][None, :, :]
    m = _silu(z1).reshape(_BI * _N, _H)
    m_ij = _silu(jnp.dot(m, we2_ref[...], preferred_element_type=f32)
                 + be2_ref[...])
    mc = _silu(jnp.dot(m_ij, wc1_ref[...], preferred_element_type=f32)
               + bc1_ref[...])
    w = jnp.dot(mc, wc2_ref[...], preferred_element_type=f32) + bc2_ref[...]

    s = w.reshape(_BI, _N, 1) / (jnp.sqrt(rad) + _EPS)
    coors = [jnp.sum(s * d[k], axis=1) for k in range(3)]
    x_new = jnp.concatenate(
        [x_i[:, k:k + 1] + coors[k] for k in range(3)], axis=1)
    x_out_ref[0] = x_new * (1.0 - _REG)

    m_i = jnp.sum(m_ij.reshape(_BI, _N, _H), axis=1)
    t1 = _silu(jnp.dot(h_i, wn1h_ref[...], preferred_element_type=f32)
               + jnp.dot(m_i, wn1m_ref[...], preferred_element_type=f32)
               + bn1_ref[...])
    t = jnp.dot(t1, wn2_ref[...], preferred_element_type=f32) + bn2_ref[...]
    h_out_ref[0] = h_i + t


def _full(shape):
    return pl.BlockSpec(shape, lambda b, i: tuple(0 for _ in shape))


def _egcl(h, x, ea4, p):
    f32 = jnp.float32
    B = h.shape[0]
    We1 = p['We1']
    w1r = We1[:_H]
    w1c = We1[_H:2 * _H]
    w1d = We1[2 * _H:2 * _H + 1]
    w1e = We1[2 * _H + 1:]
    wn1h = p['Wn1'][:_H]
    wn1m = p['Wn1'][_H:]
    weights = [
        w1r, w1c, w1d, w1e, p['be1'].reshape(1, _H),
        p['We2'], p['be2'].reshape(1, _H),
        p['Wc1'], p['bc1'].reshape(1, _H),
        p['Wc2'], p['bc2'].reshape(1, 1),
        wn1h, wn1m, p['bn1'].reshape(1, _H),
        p['Wn2'], p['bn2'].reshape(1, _H),
    ]
    in_specs = [
        pl.BlockSpec((1, _BI, _H), lambda b, i: (b, i, 0)),
        pl.BlockSpec((1, _N, _H), lambda b, i: (b, 0, 0)),
        pl.BlockSpec((1, _BI, 3), lambda b, i: (b, i, 0)),
        pl.BlockSpec((1, _N, 3), lambda b, i: (b, 0, 0)),
        pl.BlockSpec((1, _BI, _N, _EA), lambda b, i: (b, i, 0, 0)),
    ] + [_full(w.shape) for w in weights]
    out_specs = [
        pl.BlockSpec((1, _BI, _H), lambda b, i: (b, i, 0)),
        pl.BlockSpec((1, _BI, 3), lambda b, i: (b, i, 0)),
    ]
    h_out, x_out = pl.pallas_call(
        _egcl_kernel,
        grid=(B, _N // _BI),
        in_specs=in_specs,
        out_specs=out_specs,
        out_shape=[
            jax.ShapeDtypeStruct((B, _N, _H), f32),
            jax.ShapeDtypeStruct((B, _N, 3), f32),
        ],
        compiler_params=pltpu.CompilerParams(
            dimension_semantics=("parallel", "parallel")),
    )(h, h, x, x, ea4, *weights)
    return h_out, x_out


def _mlp2_kernel(x_ref, w1_ref, b1_ref, w2_ref, b2_ref, o_ref):
    f32 = jnp.float32
    z = _silu(jnp.dot(x_ref[...], w1_ref[...], preferred_element_type=f32)
              + b1_ref[...])
    o_ref[...] = jnp.dot(z, w2_ref[...], preferred_element_type=f32) + b2_ref[...]


def _mlp2(xv, W1, b1, W2, b2):
    f32 = jnp.float32
    return pl.pallas_call(
        _mlp2_kernel,
        out_shape=jax.ShapeDtypeStruct((xv.shape[0], W2.shape[1]), f32),
    )(xv, W1, b1.reshape(1, -1), W2, b2.reshape(1, -1))


def kernel(h, x, edge_attr, params):
    B = h.shape[0]
    ei = params['emb_in']
    hh = _mlp2(h.reshape(B * _N, _H), ei['W1'], ei['b1'],
               ei['W2'], ei['b2']).reshape(B, _N, _H)
    ea4 = edge_attr.reshape(B, _N, _N, _EA)
    xx = x
    for p in params['layers']:
        hh, xx = _egcl(hh, xx, ea4, p)
    eo = params['emb_out']
    return _mlp2(hh.reshape(B, _N * _H), eo['W1'], eo['b1'], eo['W2'], eo['b2'])


# speedup vs baseline: 1.0353x; 1.0353x over previous
import functools

import jax
import jax.numpy as jnp
from jax.experimental import pallas as pl
from jax.experimental.pallas import tpu as pltpu

_N = 512
_H = 64
_EA = 4
_REG = 0.01
_EPS = 1e-8
_BI = 16


def _silu(v):
    return v * jax.nn.sigmoid(v)


def _egcl_kernel(h_i_ref, h_all_ref, x_i_ref, xt_ref,
                 ea0_ref, ea1_ref, ea2_ref, ea3_ref,
                 w1r_ref, w1c_ref, w1d_ref, w1e_ref, be1_ref,
                 we2_ref, be2_ref, wc1_ref, bc1_ref, wc2_ref, bc2_ref,
                 wn1h_ref, wn1m_ref, bn1_ref, wn2_ref, bn2_ref,
                 h_out_ref, x_out_ref):
    f32 = jnp.float32
    h_i = h_i_ref[0]
    h_all = h_all_ref[0]
    x_i = x_i_ref[0]
    xt = xt_ref[0]

    hA = jnp.dot(h_i, w1r_ref[...], preferred_element_type=f32)
    hB = jnp.dot(h_all, w1c_ref[...], preferred_element_type=f32)

    d = [x_i[:, k:k + 1] - xt[k:k + 1, :] for k in range(3)]
    rad = (d[0] * d[0] + d[1] * d[1]) + d[2] * d[2]

    eas = [ea0_ref[0], ea1_ref[0], ea2_ref[0], ea3_ref[0]]
    z1 = (hA[:, None, :] + hB[None, :, :]
          + rad[:, :, None] * w1d_ref[...][None, :, :]
          + be1_ref[...][None, :, :])
    for k in range(_EA):
        z1 = z1 + eas[k][:, :, None] * w1e_ref[k:k + 1, :][None, :, :]
    m = _silu(z1).reshape(_BI * _N, _H)
    m_ij = _silu(jnp.dot(m, we2_ref[...], preferred_element_type=f32)
                 + be2_ref[...])
    mc = _silu(jnp.dot(m_ij, wc1_ref[...], preferred_element_type=f32)
               + bc1_ref[...])
    w = jnp.dot(mc, wc2_ref[...], preferred_element_type=f32) + bc2_ref[...]

    w2 = w.reshape(_BI, _N, 1)[:, :, 0]
    s = w2 / (jnp.sqrt(rad) + _EPS)
    coors = [jnp.sum(s * d[k], axis=1, keepdims=True)
             for k in range(3)]
    x_new = jnp.concatenate(
        [x_i[:, k:k + 1] + coors[k] for k in range(3)], axis=1)
    x_out_ref[0] = x_new * (1.0 - _REG)

    m_i = jnp.sum(m_ij.reshape(_BI, _N, _H), axis=1)
    t1 = _silu(jnp.dot(h_i, wn1h_ref[...], preferred_element_type=f32)
               + jnp.dot(m_i, wn1m_ref[...], preferred_element_type=f32)
               + bn1_ref[...])
    t = jnp.dot(t1, wn2_ref[...], preferred_element_type=f32) + bn2_ref[...]
    h_out_ref[0] = h_i + t


def _full(shape):
    return pl.BlockSpec(shape, lambda b, i: tuple(0 for _ in shape))


def _egcl(h, x, eas, p):
    f32 = jnp.float32
    B = h.shape[0]
    xt = jnp.swapaxes(x, 1, 2)
    We1 = p['We1']
    w1r = We1[:_H]
    w1c = We1[_H:2 * _H]
    w1d = We1[2 * _H:2 * _H + 1]
    w1e = We1[2 * _H + 1:]
    wn1h = p['Wn1'][:_H]
    wn1m = p['Wn1'][_H:]
    weights = [
        w1r, w1c, w1d, w1e, p['be1'].reshape(1, _H),
        p['We2'], p['be2'].reshape(1, _H),
        p['Wc1'], p['bc1'].reshape(1, _H),
        p['Wc2'], p['bc2'].reshape(1, 1),
        wn1h, wn1m, p['bn1'].reshape(1, _H),
        p['Wn2'], p['bn2'].reshape(1, _H),
    ]
    in_specs = [
        pl.BlockSpec((1, _BI, _H), lambda b, i: (b, i, 0)),
        pl.BlockSpec((1, _N, _H), lambda b, i: (b, 0, 0)),
        pl.BlockSpec((1, _BI, 3), lambda b, i: (b, i, 0)),
        pl.BlockSpec((1, 3, _N), lambda b, i: (b, 0, 0)),
    ] + [pl.BlockSpec((1, _BI, _N), lambda b, i: (b, i, 0))
         for _ in range(_EA)] + [_full(w.shape) for w in weights]
    out_specs = [
        pl.BlockSpec((1, _BI, _H), lambda b, i: (b, i, 0)),
        pl.BlockSpec((1, _BI, 3), lambda b, i: (b, i, 0)),
    ]
    h_out, x_out = pl.pallas_call(
        _egcl_kernel,
        grid=(B, _N // _BI),
        in_specs=in_specs,
        out_specs=out_specs,
        out_shape=[
            jax.ShapeDtypeStruct((B, _N, _H), f32),
            jax.ShapeDtypeStruct((B, _N, 3), f32),
        ],
        compiler_params=pltpu.CompilerParams(
            dimension_semantics=("parallel", "parallel")),
    )(h, h, x, xt, *eas, *weights)
    return h_out, x_out


def _mlp2_kernel(x_ref, w1_ref, b1_ref, w2_ref, b2_ref, o_ref):
    f32 = jnp.float32
    z = _silu(jnp.dot(x_ref[...], w1_ref[...], preferred_element_type=f32)
              + b1_ref[...])
    o_ref[...] = jnp.dot(z, w2_ref[...], preferred_element_type=f32) + b2_ref[...]


def _mlp2(xv, W1, b1, W2, b2):
    f32 = jnp.float32
    return pl.pallas_call(
        _mlp2_kernel,
        out_shape=jax.ShapeDtypeStruct((xv.shape[0], W2.shape[1]), f32),
    )(xv, W1, b1.reshape(1, -1), W2, b2.reshape(1, -1))


def kernel(h, x, edge_attr, params):
    B = h.shape[0]
    ei = params['emb_in']
    hh = _mlp2(h.reshape(B * _N, _H), ei['W1'], ei['b1'],
               ei['W2'], ei['b2']).reshape(B, _N, _H)
    ea4 = edge_attr.reshape(B, _N, _N, _EA)
    eas = [ea4[:, :, :, k] for k in range(_EA)]
    xx = x
    for p in params['layers']:
        hh, xx = _egcl(hh, xx, eas, p)
    eo = params['emb_out']
    return _mlp2(hh.reshape(B, _N * _H), eo['W1'], eo['b1'], eo['W2'], eo['b2'])


# speedup vs baseline: 1.0414x; 1.0059x over previous
import functools

import jax
import jax.numpy as jnp
from jax.experimental import pallas as pl
from jax.experimental.pallas import tpu as pltpu

_N = 512
_H = 64
_EA = 4
_REG = 0.01
_EPS = 1e-8
_BI = 16


def _silu(v):
    return v * jax.nn.sigmoid(v)


def _egcl_kernel(h_i_ref, h_all_ref, x_i_ref, xt_ref,
                 ea0_ref, ea1_ref, ea2_ref, ea3_ref,
                 w1r_ref, w1c_ref, w6_ref,
                 we2_ref, be2_ref, wc1_ref, bc1_ref, wc2_ref, bc2_ref,
                 wn1h_ref, wn1m_ref, bn1_ref, wn2_ref, bn2_ref,
                 h_out_ref, x_out_ref):
    f32 = jnp.float32
    h_i = h_i_ref[0]
    h_all = h_all_ref[0]
    x_i = x_i_ref[0]
    xt = xt_ref[0]

    hA = jnp.dot(h_i, w1r_ref[...], preferred_element_type=f32)
    hB = jnp.dot(h_all, w1c_ref[...], preferred_element_type=f32)

    d = [x_i[:, k:k + 1] - xt[k:k + 1, :] for k in range(3)]
    rad = (d[0] * d[0] + d[1] * d[1]) + d[2] * d[2]

    eas = [ea0_ref[0], ea1_ref[0], ea2_ref[0], ea3_ref[0]]
    fields = jnp.stack([rad] + eas + [jnp.ones_like(rad)], axis=-1)
    zb = jnp.dot(fields.reshape(_BI * _N, 6), w6_ref[...],
                 preferred_element_type=f32).reshape(_BI, _N, _H)
    z1 = zb + hA[:, None, :] + hB[None, :, :]
    m = _silu(z1).reshape(_BI * _N, _H)
    m_ij = _silu(jnp.dot(m, we2_ref[...], preferred_element_type=f32)
                 + be2_ref[...])
    mc = _silu(jnp.dot(m_ij, wc1_ref[...], preferred_element_type=f32)
               + bc1_ref[...])
    w = jnp.dot(mc, wc2_ref[...], preferred_element_type=f32) + bc2_ref[...]

    w2 = w.reshape(_BI, _N, 1)[:, :, 0]
    s = w2 / (jnp.sqrt(rad) + _EPS)
    coors = [jnp.sum(s * d[k], axis=1, keepdims=True)
             for k in range(3)]
    x_new = jnp.concatenate(
        [x_i[:, k:k + 1] + coors[k] for k in range(3)], axis=1)
    x_out_ref[0] = x_new * (1.0 - _REG)

    m_i = jnp.sum(m_ij.reshape(_BI, _N, _H), axis=1)
    t1 = _silu(jnp.dot(h_i, wn1h_ref[...], preferred_element_type=f32)
               + jnp.dot(m_i, wn1m_ref[...], preferred_element_type=f32)
               + bn1_ref[...])
    t = jnp.dot(t1, wn2_ref[...], preferred_element_type=f32) + bn2_ref[...]
    h_out_ref[0] = h_i + t


def _full(shape):
    return pl.BlockSpec(shape, lambda b, i: tuple(0 for _ in shape))


def _egcl(h, x, eas, p):
    f32 = jnp.float32
    B = h.shape[0]
    xt = jnp.swapaxes(x, 1, 2)
    We1 = p['We1']
    w1r = We1[:_H]
    w1c = We1[_H:2 * _H]
    w1d = We1[2 * _H:2 * _H + 1]
    w1e = We1[2 * _H + 1:]
    wn1h = p['Wn1'][:_H]
    wn1m = p['Wn1'][_H:]
    w6 = jnp.concatenate([w1d, w1e, p['be1'].reshape(1, _H)], axis=0)
    weights = [
        w1r, w1c, w6,
        p['We2'], p['be2'].reshape(1, _H),
        p['Wc1'], p['bc1'].reshape(1, _H),
        p['Wc2'], p['bc2'].reshape(1, 1),
        wn1h, wn1m, p['bn1'].reshape(1, _H),
        p['Wn2'], p['bn2'].reshape(1, _H),
    ]
    in_specs = [
        pl.BlockSpec((1, _BI, _H), lambda b, i: (b, i, 0)),
        pl.BlockSpec((1, _N, _H), lambda b, i: (b, 0, 0)),
        pl.BlockSpec((1, _BI, 3), lambda b, i: (b, i, 0)),
        pl.BlockSpec((1, 3, _N), lambda b, i: (b, 0, 0)),
    ] + [pl.BlockSpec((1, _BI, _N), lambda b, i: (b, i, 0))
         for _ in range(_EA)] + [_full(w.shape) for w in weights]
    out_specs = [
        pl.BlockSpec((1, _BI, _H), lambda b, i: (b, i, 0)),
        pl.BlockSpec((1, _BI, 3), lambda b, i: (b, i, 0)),
    ]
    h_out, x_out = pl.pallas_call(
        _egcl_kernel,
        grid=(B, _N // _BI),
        in_specs=in_specs,
        out_specs=out_specs,
        out_shape=[
            jax.ShapeDtypeStruct((B, _N, _H), f32),
            jax.ShapeDtypeStruct((B, _N, 3), f32),
        ],
        compiler_params=pltpu.CompilerParams(
            dimension_semantics=("parallel", "parallel")),
    )(h, h, x, xt, *eas, *weights)
    return h_out, x_out


def _mlp2_kernel(x_ref, w1_ref, b1_ref, w2_ref, b2_ref, o_ref):
    f32 = jnp.float32
    z = _silu(jnp.dot(x_ref[...], w1_ref[...], preferred_element_type=f32)
              + b1_ref[...])
    o_ref[...] = jnp.dot(z, w2_ref[...], preferred_element_type=f32) + b2_ref[...]


def _mlp2(xv, W1, b1, W2, b2):
    f32 = jnp.float32
    return pl.pallas_call(
        _mlp2_kernel,
        out_shape=jax.ShapeDtypeStruct((xv.shape[0], W2.shape[1]), f32),
    )(xv, W1, b1.reshape(1, -1), W2, b2.reshape(1, -1))


def kernel(h, x, edge_attr, params):
    B = h.shape[0]
    ei = params['emb_in']
    hh = _mlp2(h.reshape(B * _N, _H), ei['W1'], ei['b1'],
               ei['W2'], ei['b2']).reshape(B, _N, _H)
    ea4 = edge_attr.reshape(B, _N, _N, _EA)
    eas = [ea4[:, :, :, k] for k in range(_EA)]
    xx = x
    for p in params['layers']:
        hh, xx = _egcl(hh, xx, eas, p)
    eo = params['emb_out']
    return _mlp2(hh.reshape(B, _N * _H), eo['W1'], eo['b1'], eo['W2'], eo['b2'])


# speedup vs baseline: 1.0532x; 1.0113x over previous
import functools

import jax
import jax.numpy as jnp
from jax.experimental import pallas as pl
from jax.experimental.pallas import tpu as pltpu

_N = 512
_H = 64
_EA = 4
_REG = 0.01
_EPS = 1e-8
_BI = 32


def _silu(v):
    return v * jax.nn.sigmoid(v)


def _egcl_kernel(h_i_ref, h_all_ref, x_i_ref, xt_ref,
                 ea0_ref, ea1_ref, ea2_ref, ea3_ref,
                 w1r_ref, w1c_ref, w6_ref,
                 we2_ref, be2_ref, wc1_ref, bc1_ref, wc2_ref, bc2_ref,
                 wn1h_ref, wn1m_ref, bn1_ref, wn2_ref, bn2_ref,
                 h_out_ref, x_out_ref):
    f32 = jnp.float32
    h_i = h_i_ref[0]
    h_all = h_all_ref[0]
    x_i = x_i_ref[0]
    xt = xt_ref[0]

    hA = jnp.dot(h_i, w1r_ref[...], preferred_element_type=f32)
    hB = jnp.dot(h_all, w1c_ref[...], preferred_element_type=f32)

    d = [x_i[:, k:k + 1] - xt[k:k + 1, :] for k in range(3)]
    rad = (d[0] * d[0] + d[1] * d[1]) + d[2] * d[2]

    eas = [ea0_ref[0], ea1_ref[0], ea2_ref[0], ea3_ref[0]]
    fields = jnp.stack([rad] + eas + [jnp.ones_like(rad)], axis=-1)
    zb = jnp.dot(fields.reshape(_BI * _N, 6), w6_ref[...],
                 preferred_element_type=f32).reshape(_BI, _N, _H)
    z1 = zb + hA[:, None, :] + hB[None, :, :]
    m = _silu(z1).reshape(_BI * _N, _H)
    m_ij = _silu(jnp.dot(m, we2_ref[...], preferred_element_type=f32)
                 + be2_ref[...])
    mc = _silu(jnp.dot(m_ij, wc1_ref[...], preferred_element_type=f32)
               + bc1_ref[...])
    w = jnp.dot(mc, wc2_ref[...], preferred_element_type=f32) + bc2_ref[...]

    w2 = w.reshape(_BI, _N, 1)[:, :, 0]
    s = w2 / (jnp.sqrt(rad) + _EPS)
    coors = [jnp.sum(s * d[k], axis=1, keepdims=True)
             for k in range(3)]
    x_new = jnp.concatenate(
        [x_i[:, k:k + 1] + coors[k] for k in range(3)], axis=1)
    x_out_ref[0] = x_new * (1.0 - _REG)

    m_i = jnp.sum(m_ij.reshape(_BI, _N, _H), axis=1)
    t1 = _silu(jnp.dot(h_i, wn1h_ref[...], preferred_element_type=f32)
               + jnp.dot(m_i, wn1m_ref[...], preferred_element_type=f32)
               + bn1_ref[...])
    t = jnp.dot(t1, wn2_ref[...], preferred_element_type=f32) + bn2_ref[...]
    h_out_ref[0] = h_i + t


def _full(shape):
    return pl.BlockSpec(shape, lambda b, i: tuple(0 for _ in shape))


def _egcl(h, x, eas, p):
    f32 = jnp.float32
    B = h.shape[0]
    xt = jnp.swapaxes(x, 1, 2)
    We1 = p['We1']
    w1r = We1[:_H]
    w1c = We1[_H:2 * _H]
    w1d = We1[2 * _H:2 * _H + 1]
    w1e = We1[2 * _H + 1:]
    wn1h = p['Wn1'][:_H]
    wn1m = p['Wn1'][_H:]
    w6 = jnp.concatenate([w1d, w1e, p['be1'].reshape(1, _H)], axis=0)
    weights = [
        w1r, w1c, w6,
        p['We2'], p['be2'].reshape(1, _H),
        p['Wc1'], p['bc1'].reshape(1, _H),
        p['Wc2'], p['bc2'].reshape(1, 1),
        wn1h, wn1m, p['bn1'].reshape(1, _H),
        p['Wn2'], p['bn2'].reshape(1, _H),
    ]
    in_specs = [
        pl.BlockSpec((1, _BI, _H), lambda b, i: (b, i, 0)),
        pl.BlockSpec((1, _N, _H), lambda b, i: (b, 0, 0)),
        pl.BlockSpec((1, _BI, 3), lambda b, i: (b, i, 0)),
        pl.BlockSpec((1, 3, _N), lambda b, i: (b, 0, 0)),
    ] + [pl.BlockSpec((1, _BI, _N), lambda b, i: (b, i, 0))
         for _ in range(_EA)] + [_full(w.shape) for w in weights]
    out_specs = [
        pl.BlockSpec((1, _BI, _H), lambda b, i: (b, i, 0)),
        pl.BlockSpec((1, _BI, 3), lambda b, i: (b, i, 0)),
    ]
    h_out, x_out = pl.pallas_call(
        _egcl_kernel,
        grid=(B, _N // _BI),
        in_specs=in_specs,
        out_specs=out_specs,
        out_shape=[
            jax.ShapeDtypeStruct((B, _N, _H), f32),
            jax.ShapeDtypeStruct((B, _N, 3), f32),
        ],
        compiler_params=pltpu.CompilerParams(
            dimension_semantics=("parallel", "parallel")),
    )(h, h, x, xt, *eas, *weights)
    return h_out, x_out


def _mlp2_kernel(x_ref, w1_ref, b1_ref, w2_ref, b2_ref, o_ref):
    f32 = jnp.float32
    z = _silu(jnp.dot(x_ref[...], w1_ref[...], preferred_element_type=f32)
              + b1_ref[...])
    o_ref[...] = jnp.dot(z, w2_ref[...], preferred_element_type=f32) + b2_ref[...]


def _mlp2(xv, W1, b1, W2, b2):
    f32 = jnp.float32
    return pl.pallas_call(
        _mlp2_kernel,
        out_shape=jax.ShapeDtypeStruct((xv.shape[0], W2.shape[1]), f32),
    )(xv, W1, b1.reshape(1, -1), W2, b2.reshape(1, -1))


def kernel(h, x, edge_attr, params):
    B = h.shape[0]
    ei = params['emb_in']
    hh = _mlp2(h.reshape(B * _N, _H), ei['W1'], ei['b1'],
               ei['W2'], ei['b2']).reshape(B, _N, _H)
    ea4 = edge_attr.reshape(B, _N, _N, _EA)
    eas = [ea4[:, :, :, k] for k in range(_EA)]
    xx = x
    for p in params['layers']:
        hh, xx = _egcl(hh, xx, eas, p)
    eo = params['emb_out']
    return _mlp2(hh.reshape(B, _N * _H), eo['W1'], eo['b1'], eo['W2'], eo['b2'])
